# baseline (speedup 1.0000x reference)
import functools

import jax
import jax.numpy as jnp
from jax import lax
from jax.experimental import pallas as pl
from jax.experimental.pallas import tpu as pltpu
from jax.experimental.pallas import tpu_sc as plsc

_NC, _NS = 2, 16
_NW = _NC * _NS

_B, _S, _D = 4, 2048, 1024
_RPW = _B * _S // _NW


def _sc_body(tid_hbm, x_hbm, p_hbm, o_hbm, idx_v, rows_v, sem):
    c = lax.axis_index("c")
    s = lax.axis_index("s")
    wid = s * _NC + c
    wpb = _S // _RPW
    b = wid // wpb
    r0 = (wid % wpb) * _RPW
    pltpu.sync_copy(
        x_hbm.at[pl.ds(b, 1), pl.ds(r0, _RPW), :],
        o_hbm.at[pl.ds(b, 1), pl.ds(r0, _RPW), :],
    )

    @pl.when(wid == 0)
    def _():
        pltpu.sync_copy(tid_hbm, idx_v)
        pltpu.async_copy(p_hbm.at[idx_v], rows_v, sem).wait()
        for bb in range(_B):
            pltpu.sync_copy(
                rows_v.at[pl.ds(bb, 1), :],
                o_hbm.at[bb, pl.ds(_S, 1), :],
            )


@functools.partial(
    pl.kernel,
    mesh=plsc.VectorSubcoreMesh(core_axis_name="c", subcore_axis_name="s"),
    out_type=jax.ShapeDtypeStruct((_B, _S + 1, _D), jnp.float32),
    scratch_types=[
        pltpu.VMEM((_B,), jnp.int32),
        pltpu.VMEM((_B, _D), jnp.float32),
        pltpu.SemaphoreType.DMA,
    ],
)
def _sc_concat(tid_hbm, x_hbm, p_hbm, o_hbm, idx_v, rows_v, sem):
    _sc_body(tid_hbm, x_hbm, p_hbm, o_hbm, idx_v, rows_v, sem)


def kernel(x, task_id, prompt):
    B, S, D = x.shape
    task_id32 = task_id.astype(jnp.int32)
    out = _sc_concat(task_id32, x, prompt)
    return (out, task_id)


# speedup vs baseline: 11.5217x; 11.5217x over previous
import functools

import jax
import jax.numpy as jnp
from jax import lax
from jax.experimental import pallas as pl
from jax.experimental.pallas import tpu as pltpu
from jax.experimental.pallas import tpu_sc as plsc

_NC, _NS = 2, 16
_NW = _NC * _NS

_B, _S, _D = 4, 2048, 1024
_RPW = _B * _S // _NW
_CH = 32
_NCHUNK = _RPW // _CH


def _sc_body(tid_hbm, x_hbm, p_hbm, o_hbm, idx_v, rows_v, buf0, buf1, sems):
    c_ax = lax.axis_index("c")
    s_ax = lax.axis_index("s")
    wid = s_ax * _NC + c_ax
    wpb = _S // _RPW
    b = wid // wpb
    r0 = (wid % wpb) * _RPW

    bufs = (buf0, buf1)
    ins = []
    outs = []
    for c in range(_NCHUNK):
        src = x_hbm.at[pl.ds(b, 1), pl.ds(r0 + c * _CH, _CH), :]
        dst = o_hbm.at[pl.ds(b, 1), pl.ds(r0 + c * _CH, _CH), :]
        buf = bufs[c % 2]
        ins.append(pltpu.make_async_copy(src, buf, sems.at[c % 2]))
        outs.append(pltpu.make_async_copy(buf, dst, sems.at[2 + c % 2]))

    ins[0].start()
    for c in range(_NCHUNK):
        if c + 1 < _NCHUNK:
            if c >= 1:
                outs[c - 1].wait()
            ins[c + 1].start()
        ins[c].wait()
        outs[c].start()
    outs[_NCHUNK - 2].wait()
    outs[_NCHUNK - 1].wait()

    @pl.when(wid == 0)
    def _():
        pltpu.sync_copy(tid_hbm, idx_v)
        pltpu.async_copy(p_hbm.at[idx_v], rows_v, sems.at[4]).wait()
        for bb in range(_B):
            pltpu.sync_copy(
                rows_v.at[pl.ds(bb, 1), :],
                o_hbm.at[bb, pl.ds(_S, 1), :],
            )


@functools.partial(
    pl.kernel,
    mesh=plsc.VectorSubcoreMesh(core_axis_name="c", subcore_axis_name="s"),
    out_type=jax.ShapeDtypeStruct((_B, _S + 1, _D), jnp.float32),
    scratch_types=[
        pltpu.VMEM((_B,), jnp.int32),
        pltpu.VMEM((_B, _D), jnp.float32),
        pltpu.VMEM((1, _CH, _D), jnp.float32),
        pltpu.VMEM((1, _CH, _D), jnp.float32),
        pltpu.SemaphoreType.DMA((5,)),
    ],
)
def _sc_concat(tid_hbm, x_hbm, p_hbm, o_hbm, idx_v, rows_v, buf0, buf1, sems):
    _sc_body(tid_hbm, x_hbm, p_hbm, o_hbm, idx_v, rows_v, buf0, buf1, sems)


def kernel(x, task_id, prompt):
    B, S, D = x.shape
    task_id32 = task_id.astype(jnp.int32)
    out = _sc_concat(task_id32, x, prompt)
    return (out, task_id)
